# Initial kernel scaffold; baseline (speedup 1.0000x reference)
import jax
import jax.numpy as jnp
from jax.experimental import pallas as pl


def kernel(x, indices):
    raise NotImplementedError("write your pallas kernel here")


# speedup vs baseline: 5.3421x; 5.3421x over previous
import jax
import jax.numpy as jnp
from jax.experimental import pallas as pl


def _unpool_body(x_ref, idx_ref, o_ref):
    xv = x_ref[0]
    idx = idx_ref[0]
    h, w = xv.shape
    w_out = 2 * w
    ii = jax.lax.broadcasted_iota(jnp.int32, (h, w), 0)
    jj = jax.lax.broadcasted_iota(jnp.int32, (h, w), 1)
    delta = idx - (2 * ii * w_out + 2 * jj)
    r = delta >= w_out
    c = (delta & 1) == 1
    zero = jnp.zeros_like(xv)
    m00 = jnp.where(jnp.logical_and(~r, ~c), xv, zero)
    m01 = jnp.where(jnp.logical_and(~r, c), xv, zero)
    m10 = jnp.where(jnp.logical_and(r, ~c), xv, zero)
    m11 = jnp.where(jnp.logical_and(r, c), xv, zero)
    row0 = jnp.stack([m00, m01], axis=-1).reshape(h, w_out)
    row1 = jnp.stack([m10, m11], axis=-1).reshape(h, w_out)
    o_ref[0, :, :w_out] = row0
    o_ref[0, :, w_out:] = row1


def kernel(x, indices):
    b, ch, h, w = x.shape
    w_out = 2 * w
    xf = x.reshape(b * ch, h, w)
    idxf = indices.reshape(b * ch, h, w).astype(jnp.int32)
    out = pl.pallas_call(
        _unpool_body,
        grid=(b * ch,),
        in_specs=[
            pl.BlockSpec((1, h, w), lambda i: (i, 0, 0)),
            pl.BlockSpec((1, h, w), lambda i: (i, 0, 0)),
        ],
        out_specs=pl.BlockSpec((1, h, 2 * w_out), lambda i: (i, 0, 0)),
        out_shape=jax.ShapeDtypeStruct((b * ch, h, 2 * w_out), x.dtype),
    )(xf, idxf)
    return out.reshape(b, ch, 2 * h, w_out)


# speedup vs baseline: 122.0068x; 22.8387x over previous
import jax
import jax.numpy as jnp
from jax.experimental import pallas as pl


def _unpool_body(x_ref, idx_ref, e_ref, o_ref):
    xv = x_ref[0]
    idx = idx_ref[0]
    h, w = xv.shape
    w_out = 2 * w
    ii = jax.lax.broadcasted_iota(jnp.int32, (h, w), 0)
    jj = jax.lax.broadcasted_iota(jnp.int32, (h, w), 1)
    delta = idx - (2 * ii * w_out + 2 * jj)
    r = delta >= w_out
    c = (delta & 1) == 1
    zero = jnp.zeros_like(xv)
    m00 = jnp.where(jnp.logical_and(~r, ~c), xv, zero)
    m01 = jnp.where(jnp.logical_and(~r, c), xv, zero)
    m10 = jnp.where(jnp.logical_and(r, ~c), xv, zero)
    m11 = jnp.where(jnp.logical_and(r, c), xv, zero)
    m_top = jnp.concatenate([m00, m01], axis=1)
    m_bot = jnp.concatenate([m10, m11], axis=1)
    mm = jnp.concatenate([m_top, m_bot], axis=0)
    rr = jax.lax.dot_general(mm, e_ref[...],
                             (((1,), (0,)), ((), ())),
                             preferred_element_type=jnp.float32)
    o_ref[0, :, :w_out] = rr[:h]
    o_ref[0, :, w_out:] = rr[h:]


def kernel(x, indices):
    b, ch, h, w = x.shape
    w_out = 2 * w
    xf = x.reshape(b * ch, h, w)
    idxf = indices.reshape(b * ch, h, w).astype(jnp.int32)
    jr = jax.lax.broadcasted_iota(jnp.int32, (2 * w, w_out), 0)
    qc = jax.lax.broadcasted_iota(jnp.int32, (2 * w, w_out), 1)
    target = jnp.where(jr < w, 2 * jr, 2 * (jr - w) + 1)
    e = (qc == target).astype(x.dtype)
    out = pl.pallas_call(
        _unpool_body,
        grid=(b * ch,),
        in_specs=[
            pl.BlockSpec((1, h, w), lambda i: (i, 0, 0)),
            pl.BlockSpec((1, h, w), lambda i: (i, 0, 0)),
            pl.BlockSpec((2 * w, w_out), lambda i: (0, 0)),
        ],
        out_specs=pl.BlockSpec((1, h, 2 * w_out), lambda i: (i, 0, 0)),
        out_shape=jax.ShapeDtypeStruct((b * ch, h, 2 * w_out), x.dtype),
    )(xf, idxf, e)
    return out.reshape(b, ch, 2 * h, w_out)


# speedup vs baseline: 168.0030x; 1.3770x over previous
import jax
import jax.numpy as jnp
from jax.experimental import pallas as pl
from jax.experimental.pallas import tpu as pltpu

_NP = 4


def _unpool_body(x_ref, idx_ref, e_ref, o_ref):
    np_, h, w = x_ref.shape
    w_out = 2 * w
    xv = x_ref[...].reshape(np_ * h, w)
    idx = idx_ref[...].reshape(np_ * h, w)
    ii = jax.lax.broadcasted_iota(jnp.int32, (np_ * h, w), 0) % h
    jj = jax.lax.broadcasted_iota(jnp.int32, (np_ * h, w), 1)
    delta = idx - (2 * ii * w_out + 2 * jj)
    r = delta >= w_out
    c = (delta & 1) == 1
    zero = jnp.zeros_like(xv)
    m00 = jnp.where(jnp.logical_and(~r, ~c), xv, zero)
    m01 = jnp.where(jnp.logical_and(~r, c), xv, zero)
    m10 = jnp.where(jnp.logical_and(r, ~c), xv, zero)
    m11 = jnp.where(jnp.logical_and(r, c), xv, zero)
    m_top = jnp.concatenate([m00, m01], axis=1)
    m_bot = jnp.concatenate([m10, m11], axis=1)
    dims = (((1,), (0,)), ((), ()))
    r0 = jax.lax.dot_general(m_top, e_ref[...], dims,
                             preferred_element_type=jnp.float32)
    r1 = jax.lax.dot_general(m_bot, e_ref[...], dims,
                             preferred_element_type=jnp.float32)
    for p in range(np_):
        o_ref[p, :, :w_out] = r0[p * h:(p + 1) * h]
        o_ref[p, :, w_out:] = r1[p * h:(p + 1) * h]


def kernel(x, indices):
    b, ch, h, w = x.shape
    w_out = 2 * w
    xf = x.reshape(b * ch, h, w)
    idxf = indices.reshape(b * ch, h, w).astype(jnp.int32)
    jr = jax.lax.broadcasted_iota(jnp.int32, (2 * w, w_out), 0)
    qc = jax.lax.broadcasted_iota(jnp.int32, (2 * w, w_out), 1)
    target = jnp.where(jr < w, 2 * jr, 2 * (jr - w) + 1)
    e = (qc == target).astype(x.dtype)
    out = pl.pallas_call(
        _unpool_body,
        grid=(b * ch // _NP,),
        in_specs=[
            pl.BlockSpec((_NP, h, w), lambda i: (i, 0, 0)),
            pl.BlockSpec((_NP, h, w), lambda i: (i, 0, 0)),
            pl.BlockSpec((2 * w, w_out), lambda i: (0, 0)),
        ],
        out_specs=pl.BlockSpec((_NP, h, 2 * w_out), lambda i: (i, 0, 0)),
        out_shape=jax.ShapeDtypeStruct((b * ch, h, 2 * w_out), x.dtype),
        compiler_params=pltpu.CompilerParams(
            dimension_semantics=("arbitrary",)),
    )(xf, idxf, e)
    return out.reshape(b, ch, 2 * h, w_out)


# speedup vs baseline: 179.5996x; 1.0690x over previous
import jax
import jax.numpy as jnp
from jax.experimental import pallas as pl
from jax.experimental.pallas import tpu as pltpu

_NP = 8


def _unpool_body(x_ref, idx_ref, e_ref, o_ref):
    np_, h, w = x_ref.shape
    w_out = 2 * w
    xv = x_ref[...].reshape(np_ * h, w)
    idx = idx_ref[...].reshape(np_ * h, w)
    ii = jax.lax.broadcasted_iota(jnp.int32, (np_ * h, w), 0) % h
    jj = jax.lax.broadcasted_iota(jnp.int32, (np_ * h, w), 1)
    delta = idx - (2 * ii * w_out + 2 * jj)
    r = delta >= w_out
    c = (delta & 1) == 1
    zero = jnp.zeros_like(xv)
    m00 = jnp.where(jnp.logical_and(~r, ~c), xv, zero)
    m01 = jnp.where(jnp.logical_and(~r, c), xv, zero)
    m10 = jnp.where(jnp.logical_and(r, ~c), xv, zero)
    m11 = jnp.where(jnp.logical_and(r, c), xv, zero)
    m_top = jnp.concatenate([m00, m01], axis=1)
    m_bot = jnp.concatenate([m10, m11], axis=1)
    dims = (((1,), (0,)), ((), ()))
    r0 = jax.lax.dot_general(m_top, e_ref[...], dims,
                             preferred_element_type=jnp.float32)
    r1 = jax.lax.dot_general(m_bot, e_ref[...], dims,
                             preferred_element_type=jnp.float32)
    for p in range(np_):
        o_ref[p, :, :w_out] = r0[p * h:(p + 1) * h]
        o_ref[p, :, w_out:] = r1[p * h:(p + 1) * h]


def kernel(x, indices):
    b, ch, h, w = x.shape
    w_out = 2 * w
    xf = x.reshape(b * ch, h, w)
    idxf = indices.reshape(b * ch, h, w).astype(jnp.int32)
    jr = jax.lax.broadcasted_iota(jnp.int32, (2 * w, w_out), 0)
    qc = jax.lax.broadcasted_iota(jnp.int32, (2 * w, w_out), 1)
    target = jnp.where(jr < w, 2 * jr, 2 * (jr - w) + 1)
    e = (qc == target).astype(x.dtype)
    out = pl.pallas_call(
        _unpool_body,
        grid=(b * ch // _NP,),
        in_specs=[
            pl.BlockSpec((_NP, h, w), lambda i: (i, 0, 0)),
            pl.BlockSpec((_NP, h, w), lambda i: (i, 0, 0)),
            pl.BlockSpec((2 * w, w_out), lambda i: (0, 0)),
        ],
        out_specs=pl.BlockSpec((_NP, h, 2 * w_out), lambda i: (i, 0, 0)),
        out_shape=jax.ShapeDtypeStruct((b * ch, h, 2 * w_out), x.dtype),
        compiler_params=pltpu.CompilerParams(
            dimension_semantics=("arbitrary",)),
    )(xf, idxf, e)
    return out.reshape(b, ch, 2 * h, w_out)
